# Initial kernel scaffold; baseline (speedup 1.0000x reference)
import jax
import jax.numpy as jnp
from jax.experimental import pallas as pl


def kernel(inputs_batch, targets_batch):
    raise NotImplementedError("write your pallas kernel here")


# speedup vs baseline: 57.9590x; 57.9590x over previous
import functools

import jax
import jax.numpy as jnp
from jax.experimental import pallas as pl

BIG = 1e30
ROW_BLOCK = 256
KS = ((1.0, 4), (2.0, 3))


def _stats_kernel(x_ref, t_ref, pp_ref, g_ref, l_ref, *, n, nj):
    j = pl.program_id(1)
    x = x_ref[0]
    t = t_ref[0]
    r = t.shape[0]

    x2 = jnp.sum(x * x, axis=1)
    xb = x_ref[0, pl.ds(j * r, r), :]
    x2b = jnp.sum(xb * xb, axis=1)
    dist = (
        x2b[:, None]
        + x2[None, :]
        - 2.0 * jnp.dot(xb, x.T, preferred_element_type=jnp.float32,
                        precision=jax.lax.Precision.HIGHEST)
    )

    iota = jax.lax.broadcasted_iota(jnp.int32, (r, n), 1)

    mask0 = (t == 0) & (dist > 0.0)
    pv = jnp.where(mask0, dist, -BIG)
    pp = jnp.max(pv, axis=1)
    j_orig = jnp.min(jnp.where(pv == pp[:, None], iota, n), axis=1)
    cnt = jnp.sum((mask0 & (iota <= j_orig[:, None])).astype(jnp.int32), axis=1)
    g = cnt - 1
    g = jnp.where(g < 0, g + n, g)

    l_tot = jnp.zeros((r,), jnp.float32)
    for p, k in KS:
        lv = jnp.where(t == jnp.int32(p), dist, BIG)
        sel = jnp.zeros((r,), jnp.float32)
        run_m = jnp.full((r,), -jnp.inf, jnp.float32)
        run_s = jnp.zeros((r,), jnp.float32)
        for _ in range(k):
            m = jnp.min(lv, axis=1)
            eq = lv == m[:, None]
            c = jnp.sum(eq.astype(jnp.float32), axis=1)
            take = jnp.minimum(c, k - sel)
            a = jnp.abs(p - m)
            new_m = jnp.maximum(run_m, a)
            run_s = run_s * jnp.exp(run_m - new_m) + take * jnp.exp(a - new_m)
            run_m = new_m
            sel = sel + take
            lv = jnp.where(eq, BIG, lv)
        l_tot = l_tot + run_m + jnp.log(run_s)

    pp_ref[0, 0, :] = pp
    g_ref[0, 0, :] = g
    l_ref[0, 0, :] = l_tot


def _combine_kernel(pp_ref, g_ref, l_ref, out_ref, *, n, b_total):
    b = pl.program_id(0)

    @pl.when(b == 0)
    def _init():
        out_ref[...] = jnp.zeros((1, 1), jnp.float32)

    g = g_ref[0, 0, :]
    l = l_ref[0, 0, :]
    pp = pp_ref[0, 0, :]
    onehot = (g[:, None] == jax.lax.broadcasted_iota(jnp.int32, (n, n), 1))
    lg = jnp.dot(onehot.astype(jnp.float32), l[:, None],
                 preferred_element_type=jnp.float32)[:, 0]
    jv = l + lg + pp
    jv = jnp.maximum(jv, 0.0) ** 2
    out_ref[...] = out_ref[...] + jnp.sum(jv) / (n * b_total)


def kernel(inputs_batch, targets_batch):
    b, n, d = inputs_batch.shape
    r = min(ROW_BLOCK, n)
    nj = n // r

    stats = pl.pallas_call(
        functools.partial(_stats_kernel, n=n, nj=nj),
        grid=(b, nj),
        in_specs=[
            pl.BlockSpec((1, n, d), lambda bi, ji: (bi, 0, 0)),
            pl.BlockSpec((1, r, n), lambda bi, ji: (bi, ji, 0)),
        ],
        out_specs=[
            pl.BlockSpec((1, 1, r), lambda bi, ji: (bi * nj + ji, 0, 0)),
            pl.BlockSpec((1, 1, r), lambda bi, ji: (bi * nj + ji, 0, 0)),
            pl.BlockSpec((1, 1, r), lambda bi, ji: (bi * nj + ji, 0, 0)),
        ],
        out_shape=[
            jax.ShapeDtypeStruct((b * nj, 1, r), jnp.float32),
            jax.ShapeDtypeStruct((b * nj, 1, r), jnp.int32),
            jax.ShapeDtypeStruct((b * nj, 1, r), jnp.float32),
        ],
    )(inputs_batch, targets_batch)
    pp, g, l_tot = (a.reshape(b, 1, n) for a in stats)

    loss = pl.pallas_call(
        functools.partial(_combine_kernel, n=n, b_total=b),
        grid=(b,),
        in_specs=[
            pl.BlockSpec((1, 1, n), lambda bi: (bi, 0, 0)),
            pl.BlockSpec((1, 1, n), lambda bi: (bi, 0, 0)),
            pl.BlockSpec((1, 1, n), lambda bi: (bi, 0, 0)),
        ],
        out_specs=pl.BlockSpec((1, 1), lambda bi: (0, 0)),
        out_shape=jax.ShapeDtypeStruct((1, 1), jnp.float32),
    )(pp, g, l_tot)
    return loss[0, 0]
